# baseline (speedup 1.0000x reference)
import functools

import jax
import jax.numpy as jnp
from jax import lax
from jax.experimental import pallas as pl
from jax.experimental.pallas import tpu as pltpu
from jax.experimental.pallas import tpu_sc as plsc

D_MODEL = 64
EPS = 1e-5
LANES = 16
NVREG = D_MODEL // LANES
GROUP = 1024
CHUNK = 256
GATHER = 128


def _rsqrt16(v):
    bits = plsc.bitcast(v, jnp.int32)
    y = plsc.bitcast(jnp.int32(0x5F3759DF) - (bits >> 1), jnp.float32)
    half = v * 0.5
    y = y * (1.5 - half * y * y)
    y = y * (1.5 - half * y * y)
    y = y * (1.5 - half * y * y)
    return y


def _make_emb_ln(n_rows: int):
    mesh = plsc.VectorSubcoreMesh(core_axis_name="c", subcore_axis_name="s")
    nw = mesh.num_cores * mesh.num_subcores
    per_w = n_rows // nw
    assert n_rows % nw == 0 and per_w % GROUP == 0

    @functools.partial(
        pl.kernel,
        out_type=jax.ShapeDtypeStruct((n_rows // 2, 128), jnp.float32),
        mesh=mesh,
        compiler_params=pltpu.CompilerParams(
            needs_layout_passes=False, use_tc_tiling_on_sc=True),
        scratch_types=[
            pltpu.VMEM((GROUP // 128, 128), jnp.int32),
            pltpu.VMEM((GROUP // 128, 128), jnp.int32),
            pltpu.VMEM((CHUNK, 128), jnp.float32),
            pltpu.VMEM((CHUNK // 2, 128), jnp.float32),
            pltpu.VMEM((LANES, LANES), jnp.float32),
            pltpu.VMEM((LANES, LANES), jnp.float32),
            pltpu.VMEM((D_MODEL,), jnp.float32),
            pltpu.VMEM((D_MODEL,), jnp.float32),
            pltpu.SemaphoreType.DMA,
        ],
    )
    def emb_ln(idx_hbm, tab_hbm, gamma_hbm, beta_hbm, out_hbm,
               idx_v, idx2_v, rows_v, out_v, psum_v, qsum_v,
               gamma_v, beta_v, sem):
        wid = lax.axis_index("s") * mesh.num_cores + lax.axis_index("c")
        pltpu.sync_copy(gamma_hbm, gamma_v)
        pltpu.sync_copy(beta_hbm, beta_v)
        g = [gamma_v[pl.ds(c * LANES, LANES)] for c in range(NVREG)]
        b = [beta_v[pl.ds(c * LANES, LANES)] for c in range(NVREG)]
        lane = lax.iota(jnp.int32, LANES)
        idx_row0 = wid * (per_w // 128)
        out_row0 = wid * (per_w // 2)

        def group_body(grp, _):
            pltpu.sync_copy(
                idx_hbm.at[pl.ds(
                    pl.multiple_of(idx_row0 + grp * (GROUP // 128), 8),
                    GROUP // 128)],
                idx_v)
            for row in range(GROUP // 128):
                for v in range(128 // LANES):
                    idx2_v[row, pl.ds(v * LANES, LANES)] = (
                        idx_v[row, pl.ds(v * LANES, LANES)] >> 1)

            for c in range(GROUP // CHUNK):
                copies = []
                for j in range(CHUNK // GATHER):
                    copies.append(pltpu.async_copy(
                        tab_hbm.at[idx2_v.at[c * (CHUNK // GATHER) + j]],
                        rows_v.at[pl.ds(j * GATHER, GATHER)],
                        sem))
                for cp in copies:
                    cp.wait()

                def block_body(blk, _):
                    fi = c * CHUNK + blk * LANES
                    parv = idx_v[fi // 128, pl.ds(fi % 128, LANES)] & 1
                    base_row = blk * LANES
                    offs = []
                    for r16 in range(LANES):
                        offd = parv[r16] * D_MODEL
                        offs.append(offd)
                        row = base_row + r16
                        rr = [rows_v[row, pl.ds(offd + k * LANES, LANES)]
                              for k in range(NVREG)]
                        p = (rr[0] + rr[1]) + (rr[2] + rr[3])
                        q = ((rr[0] * rr[0] + rr[1] * rr[1])
                             + (rr[2] * rr[2] + rr[3] * rr[3]))
                        col = jnp.full((LANES,), r16, jnp.int32)
                        plsc.store_scatter(psum_v, [lane, col], p)
                        plsc.store_scatter(qsum_v, [lane, col], q)
                    s = psum_v[0, pl.ds(0, LANES)]
                    sq = qsum_v[0, pl.ds(0, LANES)]
                    for i in range(1, LANES):
                        s = s + psum_v[i, pl.ds(0, LANES)]
                        sq = sq + qsum_v[i, pl.ds(0, LANES)]
                    mean_v = s * (1.0 / D_MODEL)
                    var_v = sq * (1.0 / D_MODEL) - mean_v * mean_v
                    rstd_v = _rsqrt16(var_v + EPS)
                    for r16 in range(LANES):
                        row = base_row + r16
                        m = jnp.full((LANES,), mean_v[r16], jnp.float32)
                        rs = jnp.full((LANES,), rstd_v[r16], jnp.float32)
                        pair = blk * (LANES // 2) + r16 // 2
                        half = (r16 % 2) * D_MODEL
                        for k in range(NVREG):
                            a = rows_v[row, pl.ds(offs[r16] + k * LANES,
                                                  LANES)]
                            out_v[pair, pl.ds(half + k * LANES, LANES)] = (
                                (a - m) * rs * g[k] + b[k])
                    return 0

                lax.fori_loop(0, CHUNK // LANES, block_body, 0)
                pltpu.sync_copy(
                    out_v,
                    out_hbm.at[pl.ds(
                        pl.multiple_of(
                            out_row0 + grp * (GROUP // 2) + c * (CHUNK // 2),
                            8),
                        CHUNK // 2)])
            return 0

        lax.fori_loop(0, per_w // GROUP, group_body, 0)

    return emb_ln


def kernel(x, table, gamma, beta):
    bsz, seq = x.shape
    n = bsz * seq
    idx = x.reshape(n // 128, 128).astype(jnp.int32)
    tab2 = table.reshape(table.shape[0] // 2, 128)
    out = _make_emb_ln(n)(idx, tab2, gamma, beta)
    return out.reshape(bsz, seq, D_MODEL)
